# Initial kernel scaffold; baseline (speedup 1.0000x reference)
import jax
import jax.numpy as jnp
from jax.experimental import pallas as pl


def kernel(x_num, x_cat, weight, cat_table):
    raise NotImplementedError("write your pallas kernel here")


# speedup vs baseline: 1.6728x; 1.6728x over previous
import functools

import jax
import jax.numpy as jnp
import numpy as np
from jax import lax
from jax.experimental import pallas as pl
from jax.experimental.pallas import tpu as pltpu
from jax.experimental.pallas import tpu_sc as plsc

B = 4096
D_NUM = 100
N_CAT = 26
CAT_SIZE = 1000
D_TOKEN = 128
N_TOK = D_NUM + N_CAT
R = B * N_CAT

NC = 2
NS = 16
NW = NC * NS
PER_W = R // NW
CHUNK = 128
N_CHUNKS = PER_W // CHUNK

_OFFSETS = np.cumsum([0] + [CAT_SIZE] * (N_CAT - 1)).astype(np.int32)
_OFF_PATTERN = np.tile(_OFFSETS, PER_W // N_CAT).reshape(N_CHUNKS, CHUNK)


def _sc_gather_body(xcat_hbm, off_hbm, table_hbm, out_hbm, idx_v, off_v, buf0, buf1, sem0, sem1):
    w = lax.axis_index("c") * NS + lax.axis_index("s")
    base_o = w * PER_W

    pltpu.sync_copy(xcat_hbm.at[w], idx_v)
    pltpu.sync_copy(off_hbm, off_v)

    def add_offsets(r, carry):
        for i in range(CHUNK // 16):
            s = pl.ds(i * 16, 16)
            idx_v[r, s] = idx_v[r, s] + off_v[r, s]
        return carry

    lax.fori_loop(0, N_CHUNKS, add_offsets, 0)

    bufs = (buf0, buf1)
    sems = (sem0, sem1)
    copies = [None, None]
    copies[0] = pltpu.async_copy(table_hbm.at[idx_v.at[0]], bufs[0], sems[0])
    for c in range(N_CHUNKS):
        if c + 1 < N_CHUNKS:
            copies[(c + 1) % 2] = pltpu.async_copy(
                table_hbm.at[idx_v.at[c + 1]], bufs[(c + 1) % 2], sems[(c + 1) % 2])
        copies[c % 2].wait()
        pltpu.sync_copy(bufs[c % 2], out_hbm.at[pl.ds(base_o + c * CHUNK, CHUNK)])


@jax.jit
def _sc_gather(xcat2d, off2d, cat_table):
    mesh = plsc.VectorSubcoreMesh(
        core_axis_name="c", subcore_axis_name="s", num_cores=NC, num_subcores=NS)
    return pl.kernel(
        _sc_gather_body,
        out_type=jax.ShapeDtypeStruct((R, D_TOKEN), jnp.float32),
        mesh=mesh,
        scratch_types=[
            pltpu.VMEM((N_CHUNKS, CHUNK), jnp.int32),
            pltpu.VMEM((N_CHUNKS, CHUNK), jnp.int32),
            pltpu.VMEM((CHUNK, D_TOKEN), jnp.float32),
            pltpu.VMEM((CHUNK, D_TOKEN), jnp.float32),
            pltpu.SemaphoreType.DMA,
            pltpu.SemaphoreType.DMA,
        ],
    )(xcat2d, off2d, cat_table)


BB = 128


def _assemble_body(x_ref, w_ref, cat_ref, out_ref):
    out_ref[:, :D_NUM, :] = x_ref[...][:, :, None] * w_ref[...][None, :, :]
    out_ref[:, D_NUM:, :] = cat_ref[...]


@jax.jit
def _tc_assemble(x_num, weight, cat_tok):
    return pl.pallas_call(
        _assemble_body,
        grid=(B // BB,),
        in_specs=[
            pl.BlockSpec((BB, D_NUM), lambda i: (i, 0)),
            pl.BlockSpec((D_NUM, D_TOKEN), lambda i: (0, 0)),
            pl.BlockSpec((BB, N_CAT, D_TOKEN), lambda i: (i, 0, 0)),
        ],
        out_specs=pl.BlockSpec((BB, N_TOK, D_TOKEN), lambda i: (i, 0, 0)),
        out_shape=jax.ShapeDtypeStruct((B, N_TOK, D_TOKEN), jnp.float32),
    )(x_num, weight, cat_tok)


def kernel(x_num, x_cat, weight, cat_table):
    xcat2d = x_cat.reshape(NW, N_CHUNKS, CHUNK)
    off2d = jnp.asarray(_OFF_PATTERN)
    cat_flat = _sc_gather(xcat2d, off2d, cat_table)
    return _tc_assemble(x_num, weight, cat_flat.reshape(B, N_CAT, D_TOKEN))
